# baseline (speedup 1.0000x reference)
import functools

import jax
import jax.numpy as jnp
from jax import lax
from jax.experimental import pallas as pl
from jax.experimental.pallas import tpu as pltpu
from jax.experimental.pallas import tpu_sc as plsc

NBINS = 32
LANES = 16
NC = 2
NS = 16
NW = NC * NS


NBUF = 4


def _sc_body(L, D, rows_per_w, chunks, chunk_rows,
             r_hbm, m_hbm, emb_hbm, out_hbm,
             r_v, m_v, idx_v, buf, gsems, ssems):
    wid = lax.axis_index("s") * NC + lax.axis_index("c")
    row0 = wid * rows_per_w
    p0 = wid * (rows_per_w * L)
    V = emb_hbm.shape[0] // NW
    tbase = wid * V

    pltpu.sync_copy(r_hbm, r_v.at[pl.ds(0, L)])
    pltpu.sync_copy(m_hbm, m_v.at[pl.ds(0, L)])

    jchunks = L // LANES
    chunks_per_row = L // chunk_rows

    def compute_row(ri, carry):
        i = row0 + ri
        r_i = jnp.full((LANES,), 0, jnp.int32) + r_v[pl.ds(i, LANES)][0]
        m_i = jnp.full((LANES,), 0, jnp.int32) + m_v[pl.ds(i, LANES)][0]
        for jj in range(jchunks):
            rj = r_v[pl.ds(jj * LANES, LANES)]
            d = jnp.clip(rj - r_i, -NBINS, NBINS) + (NBINS + 1)
            iv = d * m_i + tbase
            c = ri * chunks_per_row + (jj * LANES) // chunk_rows
            off = (jj * LANES) % chunk_rows
            idx_v[c, pl.ds(off, LANES)] = iv
        return carry

    lax.fori_loop(0, rows_per_w, compute_row, 0)

    for b in range(NBUF):
        pltpu.async_copy(emb_hbm.at[idx_v.at[b]], buf.at[b], gsems.at[b])

    def pipe(g, carry):
        for b in range(NBUF):
            k = g * NBUF + b
            pltpu.make_async_copy(emb_hbm.at[idx_v.at[0]], buf.at[b],
                                  gsems.at[b]).wait()
            dst = out_hbm.at[pl.ds(p0 + k * chunk_rows, chunk_rows), :]
            pltpu.async_copy(buf.at[b], dst, ssems.at[b])
            pltpu.make_async_copy(buf.at[b], dst, ssems.at[b]).wait()

            @pl.when(k + NBUF < chunks)
            def _():
                pltpu.async_copy(emb_hbm.at[idx_v.at[k + NBUF]], buf.at[b],
                                 gsems.at[b])
        return carry

    lax.fori_loop(0, chunks // NBUF, pipe, 0)


def kernel(residue_index, mask, embedding):
    B, L = residue_index.shape
    V, D = embedding.shape
    r = residue_index.reshape(L).astype(jnp.int32)
    m = mask.reshape(L).astype(jnp.int32)
    emb_rep = jnp.tile(embedding, (NW, 1))

    chunk_rows = 128
    rows_per_w = L // NW
    chunks = rows_per_w * (L // chunk_rows)

    mesh = plsc.VectorSubcoreMesh(core_axis_name="c", subcore_axis_name="s",
                                  num_cores=NC, num_subcores=NS)
    body = functools.partial(_sc_body, L, D, rows_per_w, chunks, chunk_rows)
    out = pl.kernel(
        body,
        out_type=jax.ShapeDtypeStruct((L * L, D), jnp.float32),
        mesh=mesh,
        scratch_types=[
            pltpu.VMEM((L + LANES,), jnp.int32),
            pltpu.VMEM((L + LANES,), jnp.int32),
            pltpu.VMEM((chunks, chunk_rows), jnp.int32),
            pltpu.VMEM((NBUF, chunk_rows, D), jnp.float32),
            pltpu.SemaphoreType.DMA((NBUF,)),
            pltpu.SemaphoreType.DMA((NBUF,)),
        ],
    )(r, m, emb_rep)
    return out.reshape(B, L, L, D)


# speedup vs baseline: 2.0383x; 2.0383x over previous
import functools

import jax
import jax.numpy as jnp
from jax import lax
from jax.experimental import pallas as pl
from jax.experimental.pallas import tpu as pltpu
from jax.experimental.pallas import tpu_sc as plsc

NBINS = 32
LANES = 16
NC = 2
NS = 16
NW = NC * NS
NBUF = 2


def _sc_body(L, D, V, rows_per_w, chunks, chunk_rows,
             r_hbm, m_hbm, emb_hbm, out_hbm,
             r_v, m_v, table_v, buf, ssems):
    wid = lax.axis_index("s") * NC + lax.axis_index("c")
    row0 = wid * rows_per_w
    p0 = wid * (rows_per_w * L)
    cpr = L // chunk_rows
    cpr_shift = cpr.bit_length() - 1

    pltpu.sync_copy(r_hbm, r_v.at[pl.ds(0, L)])
    pltpu.sync_copy(m_hbm, m_v.at[pl.ds(0, L)])
    pltpu.sync_copy(emb_hbm, table_v)

    def pipe(g, carry):
        for b in range(NBUF):
            k = g * NBUF + b
            i = row0 + (k >> cpr_shift)
            jc = k & (cpr - 1)
            r_i = jnp.full((LANES,), 0, jnp.int32) + r_v[pl.ds(i, LANES)][0]
            m_i = jnp.full((LANES,), 0, jnp.int32) + m_v[pl.ds(i, LANES)][0]
            dst = out_hbm.at[pl.ds(p0 + k * chunk_rows, chunk_rows), :]

            @pl.when(g > 0)
            def _():
                pltpu.make_async_copy(buf.at[b], dst, ssems.at[b]).wait()

            for u in range(chunk_rows // LANES):
                rj = r_v[pl.ds(jc * chunk_rows + u * LANES, LANES)]
                d = jnp.clip(rj - r_i, -NBINS, NBINS) + (NBINS + 1)
                iv = d * m_i
                for l in range(LANES):
                    base = iv[l] * D
                    p = u * LANES + l
                    for s in range(D // LANES):
                        buf[b, p, pl.ds(s * LANES, LANES)] = (
                            table_v[pl.ds(base + s * LANES, LANES)])
            pltpu.async_copy(buf.at[b], dst, ssems.at[b])
        return carry

    lax.fori_loop(0, chunks // NBUF, pipe, 0)

    for b in range(NBUF):
        pltpu.make_async_copy(buf.at[b], out_hbm.at[pl.ds(p0, chunk_rows), :],
                              ssems.at[b]).wait()


def kernel(residue_index, mask, embedding):
    B, L = residue_index.shape
    V, D = embedding.shape
    r = residue_index.reshape(L).astype(jnp.int32)
    m = mask.reshape(L).astype(jnp.int32)
    emb_flat = embedding.reshape(V * D)

    chunk_rows = 128
    rows_per_w = L // NW
    chunks = rows_per_w * (L // chunk_rows)

    mesh = plsc.VectorSubcoreMesh(core_axis_name="c", subcore_axis_name="s",
                                  num_cores=NC, num_subcores=NS)
    body = functools.partial(_sc_body, L, D, V, rows_per_w, chunks, chunk_rows)
    out = pl.kernel(
        body,
        out_type=jax.ShapeDtypeStruct((L * L, D), jnp.float32),
        mesh=mesh,
        scratch_types=[
            pltpu.VMEM((L + LANES,), jnp.int32),
            pltpu.VMEM((L + LANES,), jnp.int32),
            pltpu.VMEM((V * D,), jnp.float32),
            pltpu.VMEM((NBUF, chunk_rows, D), jnp.float32),
            pltpu.SemaphoreType.DMA((NBUF,)),
        ],
    )(r, m, emb_flat)
    return out.reshape(B, L, L, D)


# speedup vs baseline: 5.4423x; 2.6700x over previous
import functools

import jax
import jax.numpy as jnp
from jax import lax
from jax.experimental import pallas as pl
from jax.experimental.pallas import tpu as pltpu
from jax.experimental.pallas import tpu_sc as plsc

NBINS = 32
LANES = 16
NC = 2
NS = 16
NW = NC * NS
NBUF = 2


def _sc_body(L, D, rows_per_w, chunks, chunk_rows,
             r_hbm, m_hbm, emb_hbm, out_hbm,
             r_v, m_v, table_v, idx_v, buf, gsems, ssems):
    wid = lax.axis_index("s") * NC + lax.axis_index("c")
    row0 = wid * rows_per_w
    p0 = wid * (rows_per_w * L)

    pltpu.sync_copy(r_hbm, r_v.at[pl.ds(0, L)])
    pltpu.sync_copy(m_hbm, m_v.at[pl.ds(0, L)])

    @pl.when(lax.axis_index("s") == 0)
    def _():
        pltpu.sync_copy(emb_hbm, table_v)

    plsc.subcore_barrier()

    jchunks = L // LANES
    chunks_per_row = L // chunk_rows

    def compute_row(ri, carry):
        i = row0 + ri
        r_i = jnp.full((LANES,), 0, jnp.int32) + r_v[pl.ds(i, LANES)][0]
        m_i = jnp.full((LANES,), 0, jnp.int32) + m_v[pl.ds(i, LANES)][0]
        for jj in range(jchunks):
            rj = r_v[pl.ds(jj * LANES, LANES)]
            d = jnp.clip(rj - r_i, -NBINS, NBINS) + (NBINS + 1)
            iv = d * m_i
            c = ri * chunks_per_row + (jj * LANES) // chunk_rows
            off = (jj * LANES) % chunk_rows
            idx_v[c, pl.ds(off, LANES)] = iv
        return carry

    lax.fori_loop(0, rows_per_w, compute_row, 0)

    for b in range(NBUF):
        pltpu.async_copy(table_v.at[idx_v.at[b]], buf.at[b], gsems.at[b])

    def pipe(g, carry):
        for b in range(NBUF):
            k = g * NBUF + b
            pltpu.make_async_copy(table_v.at[idx_v.at[0]], buf.at[b],
                                  gsems.at[b]).wait()
            dst = out_hbm.at[pl.ds(p0 + k * chunk_rows, chunk_rows), :]
            pltpu.async_copy(buf.at[b], dst, ssems.at[b])
            pltpu.make_async_copy(buf.at[b], dst, ssems.at[b]).wait()

            @pl.when(k + NBUF < chunks)
            def _():
                pltpu.async_copy(table_v.at[idx_v.at[k + NBUF]], buf.at[b],
                                 gsems.at[b])
        return carry

    lax.fori_loop(0, chunks // NBUF, pipe, 0)


def kernel(residue_index, mask, embedding):
    B, L = residue_index.shape
    V, D = embedding.shape
    r = residue_index.reshape(L).astype(jnp.int32)
    m = mask.reshape(L).astype(jnp.int32)

    chunk_rows = 128
    rows_per_w = L // NW
    chunks = rows_per_w * (L // chunk_rows)

    mesh = plsc.VectorSubcoreMesh(core_axis_name="c", subcore_axis_name="s",
                                  num_cores=NC, num_subcores=NS)
    body = functools.partial(_sc_body, L, D, rows_per_w, chunks, chunk_rows)
    out = pl.kernel(
        body,
        out_type=jax.ShapeDtypeStruct((L * L, D), jnp.float32),
        mesh=mesh,
        scratch_types=[
            pltpu.VMEM((L + LANES,), jnp.int32),
            pltpu.VMEM((L + LANES,), jnp.int32),
            pltpu.VMEM_SHARED((V, D), jnp.float32),
            pltpu.VMEM((chunks, chunk_rows), jnp.int32),
            pltpu.VMEM((NBUF, chunk_rows, D), jnp.float32),
            pltpu.SemaphoreType.DMA((NBUF,)),
            pltpu.SemaphoreType.DMA((NBUF,)),
        ],
    )(r, m, embedding)
    return out.reshape(B, L, L, D)
